# baseline (speedup 1.0000x reference)
import functools
import math

import jax
import jax.numpy as jnp
from jax import lax
from jax.experimental import pallas as pl
from jax.experimental.pallas import tpu as pltpu
from jax.experimental.pallas import tpu_sc as plsc

NW = 32
LANES = 16


def _splat(x, dtype=jnp.int32):
    return jnp.full((LANES,), x, dtype=dtype)


def _iota16():
    return lax.iota(jnp.int32, LANES)


def _make_repack_kernel(V, D, scale):
    W = 128
    main_rows = (V // W) * W
    nchunks = main_rows // W
    tail = V - main_rows
    niter = (nchunks + NW - 1) // NW
    mesh = plsc.VectorSubcoreMesh(core_axis_name="c", subcore_axis_name="s")

    @functools.partial(
        pl.kernel,
        mesh=mesh,
        compiler_params=pltpu.CompilerParams(
            use_tc_tiling_on_sc=True, needs_layout_passes=False),
        out_type=jax.ShapeDtypeStruct((V, 2 * D), jnp.float32),
        scratch_types=[
            pltpu.VMEM((D, W), jnp.float32),
            pltpu.VMEM((D, W), jnp.float32),
            pltpu.VMEM((W, 2 * D), jnp.float32),
            pltpu.VMEM((W, 2 * D), jnp.float32),
            pltpu.VMEM((tail, D), jnp.float32),
            pltpu.SemaphoreType.DMA,
            pltpu.SemaphoreType.DMA,
            pltpu.SemaphoreType.DMA,
            pltpu.SemaphoreType.DMA,
        ],
    )
    def k1(tableT_hbm, tail_hbm, out_hbm,
           blk0, blk1, dst0, dst1, tail_v, gs0, gs1, os0, os1):
        blk = (blk0, blk1)
        dst = (dst0, dst1)
        gs = (gs0, gs1)
        os = (os0, os1)
        wid = lax.axis_index("s") * 2 + lax.axis_index("c")

        def issue_read(ci, b):
            pltpu.async_copy(tableT_hbm.at[:, pl.ds(ci * W, W)], blk[b], gs[b])

        def wait_read(ci, b):
            pltpu.make_async_copy(
                tableT_hbm.at[:, pl.ds(ci * W, W)], blk[b], gs[b]
            ).wait()

        def drain_write(b):
            pltpu.make_async_copy(dst[b], out_hbm.at[pl.ds(0, W)], os[b]).wait()

        def compute(b):
            rows = tuple(_iota16() + LANES * g for g in range(D // LANES))

            def t_body(t, _):
                colv = _splat(t)
                for g in range(D // LANES):
                    val = plsc.load_gather(blk[b], [rows[g], colv])
                    dst[b][t, pl.ds(LANES * g, LANES)] = val * scale
                return 0

            lax.fori_loop(0, W, t_body, 0)

        issue_read(wid, 0)

        def pair_body(i2, _):
            for b in range(2):
                i = i2 * 2 + b
                ci = wid + NW * i

                @pl.when(ci < nchunks)
                def _():
                    @pl.when(ci + NW < nchunks)
                    def _():
                        issue_read(ci + NW, 1 - b)

                    wait_read(ci, b)

                    @pl.when(i >= 2)
                    def _():
                        drain_write(b)

                    compute(b)
                    pltpu.async_copy(dst[b], out_hbm.at[pl.ds(ci * W, W)], os[b])
            return 0

        lax.fori_loop(0, (niter + 1) // 2, pair_body, 0)

        for b in range(2):
            drain_write(b)

        @pl.when(wid == NW - 1)
        def _():
            pltpu.sync_copy(tail_hbm, tail_v)
            for r in range(tail):
                for g in range(D // LANES):
                    sl = pl.ds(LANES * g, LANES)
                    dst0[r, sl] = tail_v[r, sl] * scale
            pltpu.sync_copy(dst0.at[pl.ds(0, tail)],
                            out_hbm.at[pl.ds(main_rows, tail)])

    return k1


def _make_lookup_kernel(V, D, S, B, scale):
    BB = B // NW
    mesh = plsc.VectorSubcoreMesh(core_axis_name="c", subcore_axis_name="s")

    @functools.partial(
        pl.kernel,
        mesh=mesh,
        compiler_params=pltpu.CompilerParams(
            use_tc_tiling_on_sc=True, needs_layout_passes=False),
        out_type=jax.ShapeDtypeStruct((S, D, B), jnp.float32),
        scratch_types=[
            pltpu.VMEM((S, BB), jnp.int32),
            pltpu.VMEM((BB, 2 * D), jnp.float32),
            pltpu.VMEM((BB, 2 * D), jnp.float32),
            pltpu.VMEM((D, BB), jnp.float32),
            pltpu.VMEM((D, BB), jnp.float32),
            pltpu.VMEM((S, D), jnp.float32),
            pltpu.SemaphoreType.DMA,
            pltpu.SemaphoreType.DMA,
            pltpu.SemaphoreType.DMA,
            pltpu.SemaphoreType.DMA,
        ],
    )
    def k2(idxT_hbm, table8_hbm, pos_hbm, out_hbm,
           idxw, g0, g1, t0, t1, pos_v, gs0, gs1, os0, os1):
        g = (g0, g1)
        t = (t0, t1)
        gs = (gs0, gs1)
        os = (os0, os1)
        wid = lax.axis_index("s") * 2 + lax.axis_index("c")
        b0 = wid * BB

        pltpu.sync_copy(pos_hbm, pos_v)
        pltpu.sync_copy(idxT_hbm.at[:, pl.ds(b0, BB)], idxw)

        def issue_gather(s, b):
            pltpu.async_copy(table8_hbm.at[idxw.at[s]], g[b], gs[b])

        def wait_gather(s, b):
            pltpu.make_async_copy(table8_hbm.at[idxw.at[s]], g[b], gs[b]).wait()

        def drain_write(b):
            pltpu.make_async_copy(
                t[b], out_hbm.at[0, :, pl.ds(b0, BB)], os[b]
            ).wait()

        def compute(s, b):
            rows = tuple(_iota16() + LANES * j for j in range(BB // LANES))
            splat_s = _splat(s)

            def h_body(h, _):
                colv = _splat(h)
                ps = plsc.load_gather(pos_v, [splat_s, colv])
                for j in range(BB // LANES):
                    val = plsc.load_gather(g[b], [rows[j], colv])
                    t[b][h, pl.ds(LANES * j, LANES)] = val + ps
                return 0

            lax.fori_loop(0, D, h_body, 0)

        issue_gather(0, 0)

        def pair_body(s2, _):
            for b in range(2):
                s = s2 * 2 + b

                @pl.when(s + 1 < S)
                def _():
                    issue_gather(s + 1, 1 - b)

                wait_gather(s, b)

                @pl.when(s >= 2)
                def _():
                    drain_write(b)

                compute(s, b)
                pltpu.async_copy(t[b], out_hbm.at[s, :, pl.ds(b0, BB)], os[b])
            return 0

        lax.fori_loop(0, S // 2, pair_body, 0)

        for b in range(2):
            drain_write(b)

    return k2


def kernel(inputs, token_table, pos_table):
    B, S = inputs.shape
    V, D = token_table.shape
    scale = float(math.sqrt(D))
    main_rows = (V // 128) * 128

    k1 = _make_repack_kernel(V, D, scale)
    k2 = _make_lookup_kernel(V, D, S, B, scale)

    tableT = token_table.T
    tail = token_table[main_rows:]
    table8 = k1(tableT, tail)
    idxT = inputs.T.astype(jnp.int32)
    outT = k2(idxT, table8, pos_table)
    return outT.transpose(2, 0, 1)
